# baseline (speedup 1.0000x reference)
import functools

import jax
import jax.numpy as jnp
from jax import lax
from jax.experimental import pallas as pl
from jax.experimental.pallas import tpu as pltpu
from jax.experimental.pallas import tpu_sc as plsc

N = 10000
D = 128
E = 106667
NUM_REL = 3
NUM_LAYERS = 2

NC = 2
NS = 16
LANES = 16
DC = D // NC
QN = DC // LANES

K = 128
EPT_RAW = -(-E // NS)
NCHUNK = -(-EPT_RAW // K)
EPT = NCHUNK * K
E_PAD = EPT * NS

N_PAD = 10240
RN = N_PAD // NS
ZR = 32
NU = RN // ZR


def _forward(xin, rows, cols, wts):
  mesh = plsc.VectorSubcoreMesh(core_axis_name="c", subcore_axis_name="s")

  @functools.partial(
      pl.kernel,
      out_type=jax.ShapeDtypeStruct((NC * N_PAD, DC), jnp.float32),
      mesh=mesh,
      scratch_types=[
          pltpu.VMEM_SHARED((N_PAD, DC), jnp.float32),
          pltpu.VMEM_SHARED((N_PAD, DC), jnp.float32),
          pltpu.VMEM((NCHUNK, K), jnp.int32),
          pltpu.VMEM((NCHUNK, K), jnp.int32),
          pltpu.VMEM((NCHUNK, K), jnp.float32),
          pltpu.VMEM((K, DC), jnp.float32),
          pltpu.VMEM((ZR, DC), jnp.float32),
          pltpu.VMEM((ZR, DC), jnp.float32),
          pltpu.SemaphoreType.DMA,
      ],
  )
  def body(xin_hbm, rows_hbm, cols_hbm, wts_hbm, out_hbm,
           x_sp, acc_sp, rows_v, cols_v, w_v, gbuf, tmp, tmp2, sem):
    c = lax.axis_index("c")
    s = lax.axis_index("s")
    base = s * RN

    zero16 = jnp.zeros((LANES,), jnp.float32)
    zsrc = gbuf.at[pl.ds(0, ZR)]

    def zero_gbuf():
      def zero_row(i, _):
        for q in range(QN):
          gbuf[i, pl.ds(q * LANES, LANES)] = zero16
        return 0
      lax.fori_loop(0, ZR, zero_row, 0)

    zero_gbuf()
    for u in range(NU):
      off = base + u * ZR
      pltpu.sync_copy(xin_hbm.at[pl.ds(c * N_PAD + off, ZR)], tmp)
      pltpu.sync_copy(tmp, x_sp.at[pl.ds(off, ZR)])
      pltpu.sync_copy(zsrc, acc_sp.at[pl.ds(off, ZR)])
    plsc.subcore_barrier()

    for layer in range(NUM_LAYERS):
      for r in range(NUM_REL):
        pltpu.sync_copy(rows_hbm.at[r, s], rows_v)
        pltpu.sync_copy(cols_hbm.at[r, s], cols_v)
        pltpu.sync_copy(wts_hbm.at[r, s], w_v)

        def chunk(j, _):
          pltpu.async_copy(x_sp.at[cols_v.at[j]], gbuf, sem).wait()

          def scale16(b, _):
            w16v = w_v[j, pl.ds(b * LANES, LANES)]
            for e in range(LANES):
              row = b * LANES + e
              we = jnp.full((LANES,), w16v[e], jnp.float32)
              for q in range(QN):
                sl = pl.ds(q * LANES, LANES)
                gbuf[row, sl] = gbuf[row, sl] * we
            return 0
          lax.fori_loop(0, K // LANES, scale16, 0)

          pltpu.sync_copy(gbuf, acc_sp.at[rows_v.at[j]], add=True)
          return 0
        lax.fori_loop(0, NCHUNK, chunk, 0)
        plsc.subcore_barrier()

        zero_gbuf()
        for u in range(NU):
          off = base + u * ZR
          pltpu.sync_copy(acc_sp.at[pl.ds(off, ZR)], tmp)
          pltpu.sync_copy(zsrc, acc_sp.at[pl.ds(off, ZR)])
          if r > 0:
            pltpu.sync_copy(out_hbm.at[pl.ds(c * N_PAD + off, ZR)], tmp2)

          def leaky(i, _):
            for q in range(QN):
              sl = pl.ds(q * LANES, LANES)
              v = tmp[i, sl]
              lv = jnp.maximum(v, v * 0.01)
              if r == 0:
                tmp[i, sl] = lv
              else:
                tmp2[i, sl] = tmp2[i, sl] + lv
            return 0
          lax.fori_loop(0, ZR, leaky, 0)

          src = tmp if r == 0 else tmp2
          pltpu.sync_copy(src, out_hbm.at[pl.ds(c * N_PAD + off, ZR)])
        plsc.subcore_barrier()

      if layer < NUM_LAYERS - 1:
        for u in range(NU):
          off = base + u * ZR
          pltpu.sync_copy(out_hbm.at[pl.ds(c * N_PAD + off, ZR)], tmp)
          pltpu.sync_copy(tmp, x_sp.at[pl.ds(off, ZR)])
        plsc.subcore_barrier()

  return body(xin, rows, cols, wts)


def _prep_edges(edge_index, edge_weight):
  rows = edge_index[0]
  cols = edge_index[1]
  pad = E_PAD - E
  rows = jnp.concatenate([rows, jnp.zeros((pad,), rows.dtype)])
  cols = jnp.concatenate([cols, jnp.zeros((pad,), cols.dtype)])
  w = jnp.concatenate([edge_weight, jnp.zeros((pad,), edge_weight.dtype)])
  return (rows.reshape(NS, NCHUNK, K), cols.reshape(NS, NCHUNK, K),
          w.reshape(NS, NCHUNK, K))


@jax.jit
def kernel(init_embs, edge_index_r0, edge_weight_r0, edge_index_r1,
           edge_weight_r1, edge_index_r2, edge_weight_r2):
  xpad = jnp.concatenate(
      [init_embs, jnp.zeros((N_PAD - N, D), init_embs.dtype)])
  xin = jnp.concatenate([xpad[:, :DC], xpad[:, DC:]])
  r0 = _prep_edges(edge_index_r0, edge_weight_r0)
  r1 = _prep_edges(edge_index_r1, edge_weight_r1)
  r2 = _prep_edges(edge_index_r2, edge_weight_r2)
  rows = jnp.stack([r0[0], r1[0], r2[0]])
  cols = jnp.stack([r0[1], r1[1], r2[1]])
  wts = jnp.stack([r0[2], r1[2], r2[2]])
  out = _forward(xin, rows, cols, wts)
  return jnp.concatenate([out[:N], out[N_PAD:N_PAD + N]], axis=1)
